# Initial kernel scaffold; baseline (speedup 1.0000x reference)
import jax
import jax.numpy as jnp
from jax.experimental import pallas as pl


def kernel(pos, x, o, W, bn_gamma, bn_beta):
    raise NotImplementedError("write your pallas kernel here")


# speedup vs baseline: 10.8488x; 10.8488x over previous
import functools

import jax
import jax.numpy as jnp
from jax import lax
from jax.experimental import pallas as pl
from jax.experimental.pallas import tpu as pltpu
from jax.experimental.pallas import tpu_sc as plsc

N = 16384
C_IN = 64
C_OUT = 128
M = 4096
KNN = 16
BM = 256
NW = 32
SEG_PER_W = M // NW
ROWS_PER_W = SEG_PER_W * KNN
F32 = jnp.float32
I32 = jnp.int32


def _fps_body(px_ref, py_ref, pz_ref, qpad_ref):
    px = px_ref[...]
    py = py_ref[...]
    pz = pz_ref[...]
    row_i = lax.broadcasted_iota(I32, (128, 128), 0)
    col_i = lax.broadcasted_iota(I32, (128, 128), 1)
    flat = row_i * 128 + col_i
    lane = lax.broadcasted_iota(I32, (1, 128), 1)
    big_i = jnp.int32(2 ** 30)

    def emit_row(i, qx, qy, qz):
        r = jnp.where(lane == 0, qx,
                      jnp.where(lane == 1, qy,
                                jnp.where(lane == 2, qz, 0.0))).astype(F32)
        qpad_ref[pl.ds(i, 1), :] = r

    m0 = flat == 0
    qx0 = jnp.sum(jnp.where(m0, px, 0.0))
    qy0 = jnp.sum(jnp.where(m0, py, 0.0))
    qz0 = jnp.sum(jnp.where(m0, pz, 0.0))
    emit_row(0, qx0, qy0, qz0)

    def body(i, carry):
        min_d, qx, qy, qz = carry
        dx = px - qx
        dy = py - qy
        dz = pz - qz
        d = (dx * dx + dz * dz) + dy * dy
        min_d = jnp.minimum(min_d, d)
        mx = jnp.max(min_d)
        idx = jnp.min(jnp.where(min_d == mx, flat, big_i))
        msk = flat == idx
        nqx = jnp.sum(jnp.where(msk, px, 0.0))
        nqy = jnp.sum(jnp.where(msk, py, 0.0))
        nqz = jnp.sum(jnp.where(msk, pz, 0.0))
        emit_row(i, nqx, nqy, nqz)
        return (min_d, nqx, nqy, nqz)

    init = (jnp.full((128, 128), jnp.inf, F32), qx0, qy0, qz0)
    lax.fori_loop(1, M, body, init)


def _fps_call(px, py, pz):
    return pl.pallas_call(
        _fps_body,
        out_shape=jax.ShapeDtypeStruct((M, 128), F32),
    )(px, py, pz)


CW = 1024
NCH = N // CW


def _knn_body(qpad_ref, pos_ref, pp_ref, nbrt_ref, d2t_ref):
    q = qpad_ref[...][:, :3]
    prod = lax.dot_general(pos_ref[...], q, (((1,), (1,)), ((), ())),
                           preferred_element_type=F32)
    d2t_ref[...] = pp_ref[...] - 2.0 * prod

    big_i = jnp.int32(2 ** 30)
    inf = jnp.float32(jnp.inf)
    sub = lax.broadcasted_iota(I32, (CW, BM), 0)

    def round_body(k, carry):
        dprev, iprev = carry

        def chunk_body(c, mc):
            mn, mi = mc
            blk = d2t_ref[pl.ds(c * CW, CW), :]
            gidx = sub + c * CW
            valid = (blk > dprev) | ((blk == dprev) & (gidx > iprev))
            cv = jnp.where(valid, blk, inf)
            bmn = jnp.min(cv, axis=0, keepdims=True)
            bmi = jnp.min(jnp.where(cv == bmn, gidx, big_i), axis=0,
                          keepdims=True)
            take = bmn < mn
            return (jnp.where(take, bmn, mn), jnp.where(take, bmi, mi))

        mn0 = jnp.full((1, BM), jnp.inf, F32)
        mi0 = jnp.full((1, BM), big_i, I32)
        mn, mi = lax.fori_loop(0, NCH, chunk_body, (mn0, mi0))
        nbrt_ref[pl.ds(k, 1), :] = mi
        return (mn, mi)

    init = (jnp.full((1, BM), -jnp.inf, F32), jnp.full((1, BM), -1, I32))
    lax.fori_loop(0, KNN, round_body, init)


def _knn_call(qpad, pos, pp):
    return pl.pallas_call(
        _knn_body,
        grid=(M // BM,),
        in_specs=[
            pl.BlockSpec((BM, 128), lambda i: (i, 0)),
            pl.BlockSpec((N, 3), lambda i: (0, 0)),
            pl.BlockSpec((N, 1), lambda i: (0, 0)),
        ],
        out_specs=pl.BlockSpec((KNN, BM), lambda i: (0, i)),
        out_shape=jax.ShapeDtypeStruct((KNN, M), I32),
        scratch_shapes=[pltpu.VMEM((N, BM), F32)],
    )(qpad, pos, pp)


def _g_body(pxc_ref, wt_ref, g_ref):
    g_ref[...] = jnp.dot(pxc_ref[...], wt_ref[...], preferred_element_type=F32)


def _g_call(pxc, wt):
    return pl.pallas_call(
        _g_body,
        out_shape=jax.ShapeDtypeStruct((N, C_OUT), F32),
    )(pxc, wt)


def _seg_body(g_hbm, col_hbm, omax, omin, osum, osumq,
              idx_v, rows_v, smax, smin, ssum, ssumq, sem):
    nc = 2
    wid = lax.axis_index("s") * nc + lax.axis_index("c")
    base = wid * ROWS_PER_W

    def chunk_body(ch, _):
        pltpu.sync_copy(col_hbm.at[pl.ds(base + ch * 128, 128)], idx_v)
        pltpu.async_copy(g_hbm.at[idx_v], rows_v, sem).wait()

        def seg_body(sj, _):
            def ch_group(c, _):
                co = c * 16
                r0 = sj * 16
                v = rows_v[r0, pl.ds(co, 16)]
                mx = v
                mn = v
                sm = v
                sq = v * v
                for j in range(1, KNN):
                    v = rows_v[r0 + j, pl.ds(co, 16)]
                    mx = jnp.maximum(mx, v)
                    mn = jnp.minimum(mn, v)
                    sm = sm + v
                    sq = sq + v * v
                seg = ch * 8 + sj
                smax[seg, pl.ds(co, 16)] = mx
                smin[seg, pl.ds(co, 16)] = mn
                ssum[seg, pl.ds(co, 16)] = sm
                ssumq[seg, pl.ds(co, 16)] = sq
                return _

            return lax.fori_loop(0, 8, ch_group, None)

        lax.fori_loop(0, 8, seg_body, None)
        return _

    lax.fori_loop(0, 16, chunk_body, None)
    ob = wid * SEG_PER_W
    pltpu.sync_copy(smax, omax.at[pl.ds(ob, SEG_PER_W)])
    pltpu.sync_copy(smin, omin.at[pl.ds(ob, SEG_PER_W)])
    pltpu.sync_copy(ssum, osum.at[pl.ds(ob, SEG_PER_W)])
    pltpu.sync_copy(ssumq, osumq.at[pl.ds(ob, SEG_PER_W)])


def _seg_call(g, col):
    mesh = plsc.VectorSubcoreMesh(core_axis_name="c", subcore_axis_name="s")
    f = functools.partial(
        pl.kernel,
        mesh=mesh,
        out_type=[jax.ShapeDtypeStruct((M, C_OUT), F32)] * 4,
        scratch_types=[
            pltpu.VMEM((128,), I32),
            pltpu.VMEM((128, C_OUT), F32),
            pltpu.VMEM((SEG_PER_W, C_OUT), F32),
            pltpu.VMEM((SEG_PER_W, C_OUT), F32),
            pltpu.VMEM((SEG_PER_W, C_OUT), F32),
            pltpu.VMEM((SEG_PER_W, C_OUT), F32),
            pltpu.SemaphoreType.DMA,
        ],
    )(_seg_body)
    return f(g, col)


def _fin_body(qpad_ref, w13_ref, maxg_ref, ming_ref, sumg_ref, sumq_ref,
              gamma_ref, beta_ref, out_ref):
    q = qpad_ref[...][:, :3]
    qw = jnp.dot(q, w13_ref[...], preferred_element_type=F32)
    maxh = maxg_ref[...] - qw
    minh = ming_ref[...] - qw
    s = sumg_ref[...]
    sq = sumq_ref[...]
    kf = jnp.float32(KNN)
    ntot = jnp.float32(M * KNN)
    tot = jnp.sum(s - kf * qw, axis=0, keepdims=True)
    mean = tot / ntot
    totsq = jnp.sum(sq - 2.0 * qw * s + kf * qw * qw, axis=0, keepdims=True)
    var = totsq / ntot - mean * mean
    rstd = lax.rsqrt(var + 1e-5)
    scale = gamma_ref[...] * rstd
    shift = beta_ref[...] - mean * scale
    ext = jnp.where(scale >= 0.0, maxh, minh)
    out_ref[...] = jnp.maximum(ext * scale + shift, 0.0)


def _fin_call(qpad, w13, omax, omin, osum, osumq, gamma, beta):
    return pl.pallas_call(
        _fin_body,
        out_shape=jax.ShapeDtypeStruct((M, C_OUT), F32),
    )(qpad, w13, omax, omin, osum, osumq, gamma, beta)


def kernel(pos, x, o, W, bn_gamma, bn_beta):
    pos = pos.astype(F32)
    x = x.astype(F32)
    px = pos[:, 0].reshape(128, 128)
    py = pos[:, 1].reshape(128, 128)
    pz = pos[:, 2].reshape(128, 128)
    qpad = _fps_call(px, py, pz)
    pp = jnp.sum(pos * pos, axis=1, keepdims=True)
    nbrt = _knn_call(qpad, pos, pp)
    pxc = jnp.concatenate([pos, x], axis=1)
    g = _g_call(pxc, W.T.astype(F32))
    col = nbrt.T.reshape(-1)
    omax, omin, osum, osumq = _seg_call(g, col)
    out = _fin_call(qpad, W[:, :3].T.astype(F32), omax, omin, osum, osumq,
                    bn_gamma.reshape(1, C_OUT), bn_beta.reshape(1, C_OUT))
    return (qpad[:, :3], out)


# speedup vs baseline: 12.6422x; 1.1653x over previous
import functools

import jax
import jax.numpy as jnp
from jax import lax
from jax.experimental import pallas as pl
from jax.experimental.pallas import tpu as pltpu
from jax.experimental.pallas import tpu_sc as plsc

N = 16384
C_IN = 64
C_OUT = 128
M = 4096
KNN = 16
BM = 256
NW = 32
SEG_PER_W = M // NW
ROWS_PER_W = SEG_PER_W * KNN
F32 = jnp.float32
I32 = jnp.int32


def _fps_body(pxs_ref, pys_ref, pzs_ref, px_ref, py_ref, pz_ref, qpad_ref):
    px = px_ref[...]
    py = py_ref[...]
    pz = pz_ref[...]
    row_i = lax.broadcasted_iota(I32, (128, 128), 0)
    col_i = lax.broadcasted_iota(I32, (128, 128), 1)
    flat = row_i * 128 + col_i
    lane = lax.broadcasted_iota(I32, (1, 128), 1)
    big_i = jnp.int32(2 ** 30)

    def emit_row(i, qx, qy, qz):
        r = jnp.where(lane == 0, qx,
                      jnp.where(lane == 1, qy,
                                jnp.where(lane == 2, qz, 0.0))).astype(F32)
        qpad_ref[pl.ds(i, 1), :] = r

    qx0 = pxs_ref[0]
    qy0 = pys_ref[0]
    qz0 = pzs_ref[0]
    emit_row(0, qx0, qy0, qz0)

    def body(i, carry):
        min_d, qx, qy, qz = carry
        dx = px - qx
        dy = py - qy
        dz = pz - qz
        d = (dx * dx + dz * dz) + dy * dy
        min_d = jnp.minimum(min_d, d)
        mx = jnp.max(min_d)
        idx = jnp.min(jnp.where(min_d == mx, flat, big_i))
        nqx = pxs_ref[idx]
        nqy = pys_ref[idx]
        nqz = pzs_ref[idx]
        emit_row(i, nqx, nqy, nqz)
        return (min_d, nqx, nqy, nqz)

    init = (jnp.full((128, 128), jnp.inf, F32), qx0, qy0, qz0)
    lax.fori_loop(1, M, body, init)


def _fps_call(px, py, pz):
    smem = pl.BlockSpec(memory_space=pltpu.SMEM)
    return pl.pallas_call(
        _fps_body,
        in_specs=[smem, smem, smem,
                  pl.BlockSpec((128, 128), lambda: (0, 0)),
                  pl.BlockSpec((128, 128), lambda: (0, 0)),
                  pl.BlockSpec((128, 128), lambda: (0, 0))],
        out_shape=jax.ShapeDtypeStruct((M, 128), F32),
    )(px.reshape(N), py.reshape(N), pz.reshape(N), px, py, pz)


CW = 1024
NCH = N // CW


def _knn_body(qpad_ref, pos_ref, pp_ref, nbrt_ref, d2t_ref):
    q = qpad_ref[...][:, :3]
    prod = lax.dot_general(pos_ref[...], q, (((1,), (1,)), ((), ())),
                           preferred_element_type=F32)
    d2t_ref[...] = pp_ref[...] - 2.0 * prod

    big_i = jnp.int32(2 ** 30)
    inf = jnp.float32(jnp.inf)
    sub = lax.broadcasted_iota(I32, (CW, BM), 0)

    def round_body(k, carry):
        dprev, iprev = carry

        def chunk_body(c, mc):
            mn, mi = mc
            blk = d2t_ref[pl.ds(c * CW, CW), :]
            gidx = sub + c * CW
            valid = (blk > dprev) | ((blk == dprev) & (gidx > iprev))
            cv = jnp.where(valid, blk, inf)
            bmn = jnp.min(cv, axis=0, keepdims=True)
            bmi = jnp.min(jnp.where(cv == bmn, gidx, big_i), axis=0,
                          keepdims=True)
            take = bmn < mn
            return (jnp.where(take, bmn, mn), jnp.where(take, bmi, mi))

        mn0 = jnp.full((1, BM), jnp.inf, F32)
        mi0 = jnp.full((1, BM), big_i, I32)
        mn, mi = lax.fori_loop(0, NCH, chunk_body, (mn0, mi0))
        nbrt_ref[pl.ds(k, 1), :] = mi
        return (mn, mi)

    init = (jnp.full((1, BM), -jnp.inf, F32), jnp.full((1, BM), -1, I32))
    lax.fori_loop(0, KNN, round_body, init)


def _knn_call(qpad, pos, pp):
    return pl.pallas_call(
        _knn_body,
        grid=(M // BM,),
        in_specs=[
            pl.BlockSpec((BM, 128), lambda i: (i, 0)),
            pl.BlockSpec((N, 3), lambda i: (0, 0)),
            pl.BlockSpec((N, 1), lambda i: (0, 0)),
        ],
        out_specs=pl.BlockSpec((KNN, BM), lambda i: (0, i)),
        out_shape=jax.ShapeDtypeStruct((KNN, M), I32),
        scratch_shapes=[pltpu.VMEM((N, BM), F32)],
    )(qpad, pos, pp)


def _g_body(pxc_ref, wt_ref, g_ref):
    g_ref[...] = jnp.dot(pxc_ref[...], wt_ref[...], preferred_element_type=F32)


def _g_call(pxc, wt):
    return pl.pallas_call(
        _g_body,
        out_shape=jax.ShapeDtypeStruct((N, C_OUT), F32),
    )(pxc, wt)


def _seg_body(g_hbm, col_hbm, omax, omin, osum, osumq,
              idx_v, rows_v, smax, smin, ssum, ssumq, sem):
    nc = 2
    wid = lax.axis_index("s") * nc + lax.axis_index("c")
    base = wid * ROWS_PER_W

    def chunk_body(ch, _):
        pltpu.sync_copy(col_hbm.at[pl.ds(base + ch * 128, 128)], idx_v)
        pltpu.async_copy(g_hbm.at[idx_v], rows_v, sem).wait()

        def seg_body(sj, _):
            def ch_group(c, _):
                co = c * 16
                r0 = sj * 16
                v = rows_v[r0, pl.ds(co, 16)]
                mx = v
                mn = v
                sm = v
                sq = v * v
                for j in range(1, KNN):
                    v = rows_v[r0 + j, pl.ds(co, 16)]
                    mx = jnp.maximum(mx, v)
                    mn = jnp.minimum(mn, v)
                    sm = sm + v
                    sq = sq + v * v
                seg = ch * 8 + sj
                smax[seg, pl.ds(co, 16)] = mx
                smin[seg, pl.ds(co, 16)] = mn
                ssum[seg, pl.ds(co, 16)] = sm
                ssumq[seg, pl.ds(co, 16)] = sq
                return _

            return lax.fori_loop(0, 8, ch_group, None)

        lax.fori_loop(0, 8, seg_body, None)
        return _

    lax.fori_loop(0, 16, chunk_body, None)
    ob = wid * SEG_PER_W
    pltpu.sync_copy(smax, omax.at[pl.ds(ob, SEG_PER_W)])
    pltpu.sync_copy(smin, omin.at[pl.ds(ob, SEG_PER_W)])
    pltpu.sync_copy(ssum, osum.at[pl.ds(ob, SEG_PER_W)])
    pltpu.sync_copy(ssumq, osumq.at[pl.ds(ob, SEG_PER_W)])


def _seg_call(g, col):
    mesh = plsc.VectorSubcoreMesh(core_axis_name="c", subcore_axis_name="s")
    f = functools.partial(
        pl.kernel,
        mesh=mesh,
        out_type=[jax.ShapeDtypeStruct((M, C_OUT), F32)] * 4,
        scratch_types=[
            pltpu.VMEM((128,), I32),
            pltpu.VMEM((128, C_OUT), F32),
            pltpu.VMEM((SEG_PER_W, C_OUT), F32),
            pltpu.VMEM((SEG_PER_W, C_OUT), F32),
            pltpu.VMEM((SEG_PER_W, C_OUT), F32),
            pltpu.VMEM((SEG_PER_W, C_OUT), F32),
            pltpu.SemaphoreType.DMA,
        ],
    )(_seg_body)
    return f(g, col)


def _fin_body(qpad_ref, w13_ref, maxg_ref, ming_ref, sumg_ref, sumq_ref,
              gamma_ref, beta_ref, out_ref):
    q = qpad_ref[...][:, :3]
    qw = jnp.dot(q, w13_ref[...], preferred_element_type=F32)
    maxh = maxg_ref[...] - qw
    minh = ming_ref[...] - qw
    s = sumg_ref[...]
    sq = sumq_ref[...]
    kf = jnp.float32(KNN)
    ntot = jnp.float32(M * KNN)
    tot = jnp.sum(s - kf * qw, axis=0, keepdims=True)
    mean = tot / ntot
    totsq = jnp.sum(sq - 2.0 * qw * s + kf * qw * qw, axis=0, keepdims=True)
    var = totsq / ntot - mean * mean
    rstd = lax.rsqrt(var + 1e-5)
    scale = gamma_ref[...] * rstd
    shift = beta_ref[...] - mean * scale
    ext = jnp.where(scale >= 0.0, maxh, minh)
    out_ref[...] = jnp.maximum(ext * scale + shift, 0.0)


def _fin_call(qpad, w13, omax, omin, osum, osumq, gamma, beta):
    return pl.pallas_call(
        _fin_body,
        out_shape=jax.ShapeDtypeStruct((M, C_OUT), F32),
    )(qpad, w13, omax, omin, osum, osumq, gamma, beta)


def kernel(pos, x, o, W, bn_gamma, bn_beta):
    pos = pos.astype(F32)
    x = x.astype(F32)
    px = pos[:, 0].reshape(128, 128)
    py = pos[:, 1].reshape(128, 128)
    pz = pos[:, 2].reshape(128, 128)
    qpad = _fps_call(px, py, pz)
    pp = jnp.sum(pos * pos, axis=1, keepdims=True)
    nbrt = _knn_call(qpad, pos, pp)
    pxc = jnp.concatenate([pos, x], axis=1)
    g = _g_call(pxc, W.T.astype(F32))
    col = nbrt.T.reshape(-1)
    omax, omin, osum, osumq = _seg_call(g, col)
    out = _fin_call(qpad, W[:, :3].T.astype(F32), omax, omin, osum, osumq,
                    bn_gamma.reshape(1, C_OUT), bn_beta.reshape(1, C_OUT))
    return (qpad[:, :3], out)
